# Initial kernel scaffold; baseline (speedup 1.0000x reference)
import jax
import jax.numpy as jnp
from jax.experimental import pallas as pl


def kernel(x, table):
    raise NotImplementedError("write your pallas kernel here")


# speedup vs baseline: 5.0505x; 5.0505x over previous
import functools

import jax
import jax.numpy as jnp
from jax import lax
from jax.experimental import pallas as pl
from jax.experimental.pallas import tpu as pltpu
from jax.experimental.pallas import tpu_sc as plsc

_L = 128
_K = 8
_C = _L * _K


@functools.cache
def _gather_fn(V, D, N):
  info = plsc.get_sparse_core_info()
  NC, NS = info.num_cores, info.num_subcores
  NW = NC * NS
  RPW = N // NW
  assert N % NW == 0 and RPW % _C == 0
  G = RPW // _C
  mesh = plsc.VectorSubcoreMesh(core_axis_name="c", subcore_axis_name="s")

  @functools.partial(
      pl.kernel, mesh=mesh,
      out_type=jax.ShapeDtypeStruct((N, D), jnp.float32),
      compiler_params=pltpu.CompilerParams(use_tc_tiling_on_sc=False),
      scratch_types=[
          pltpu.VMEM((_K, _L), jnp.int32),
          pltpu.VMEM((_C, D), jnp.float32),
          pltpu.SemaphoreType.DMA,
      ],
  )
  def k(table_hbm, idx_hbm, out_hbm, idx_v, rows_v, sem):
    wid = lax.axis_index("s") * NC + lax.axis_index("c")
    irow0 = wid * (RPW // _L)

    def body(g, carry):
      r = irow0 + g * _K
      pltpu.sync_copy(idx_hbm.at[pl.ds(r, _K)], idx_v)
      copies = [
          pltpu.async_copy(table_hbm.at[idx_v.at[j]],
                           rows_v.at[pl.ds(j * _L, _L)], sem)
          for j in range(_K)
      ]
      for c in copies:
        c.wait()
      pltpu.sync_copy(rows_v, out_hbm.at[pl.ds(r * _L, _C)])
      return carry

    lax.fori_loop(0, G, body, 0)

  return k


def kernel(x, table):
  B, H = x.shape
  V, D = table.shape
  N = B * H
  idx2d = x.reshape(N // _L, _L)
  out = _gather_fn(V, D, N)(table, idx2d)
  return out.reshape(B, H, D)


# speedup vs baseline: 5.2642x; 1.0423x over previous
import functools

import jax
import jax.numpy as jnp
from jax import lax
from jax.experimental import pallas as pl
from jax.experimental.pallas import tpu as pltpu
from jax.experimental.pallas import tpu_sc as plsc

_L = 128
_K = 10
_C = _L * _K


@functools.cache
def _gather_fn(V, D, N):
  info = plsc.get_sparse_core_info()
  NC, NS = info.num_cores, info.num_subcores
  NW = NC * NS
  RPW = N // NW
  assert N % NW == 0 and RPW % _C == 0
  G = RPW // _C
  assert G >= 4 and G % 2 == 0
  mesh = plsc.VectorSubcoreMesh(core_axis_name="c", subcore_axis_name="s")

  @functools.partial(
      pl.kernel, mesh=mesh,
      out_type=jax.ShapeDtypeStruct((N, D), jnp.float32),
      compiler_params=pltpu.CompilerParams(use_tc_tiling_on_sc=False),
      scratch_types=[
          pltpu.VMEM((2, _K, _L), jnp.int32),
          pltpu.VMEM((2, _C, D), jnp.float32),
          pltpu.SemaphoreType.DMA,
          pltpu.SemaphoreType.DMA,
          pltpu.SemaphoreType.DMA,
          pltpu.SemaphoreType.DMA,
          pltpu.SemaphoreType.DMA,
          pltpu.SemaphoreType.DMA,
      ],
  )
  def k(table_hbm, idx_hbm, out_hbm, idx_v, rows_v,
        isem0, isem1, gsem0, gsem1, wsem0, wsem1):
    isems, gsems, wsems = (isem0, isem1), (gsem0, gsem1), (wsem0, wsem1)
    wid = lax.axis_index("s") * NC + lax.axis_index("c")
    irow0 = wid * (RPW // _L)

    def idx_start(h, b):
      r = jnp.where(h < G, irow0 + h * _K, irow0)
      pltpu.async_copy(idx_hbm.at[pl.ds(r, _K)], idx_v.at[b], isems[b])

    def idx_wait(b):
      pltpu.make_async_copy(
          idx_hbm.at[pl.ds(irow0, _K)], idx_v.at[b], isems[b]).wait()

    def fire(b):
      for j in range(_K):
        pltpu.async_copy(table_hbm.at[idx_v.at[b, j]],
                         rows_v.at[b, pl.ds(j * _L, _L)], gsems[b])

    def drain(b):
      for j in range(_K):
        pltpu.make_async_copy(table_hbm.at[idx_v.at[b, j]],
                              rows_v.at[b, pl.ds(j * _L, _L)], gsems[b]).wait()

    def wb_start(g, b):
      pltpu.async_copy(
          rows_v.at[b], out_hbm.at[pl.ds((irow0 + g * _K) * _L, _C)], wsems[b])

    def wb_wait(b):
      pltpu.make_async_copy(
          rows_v.at[b], out_hbm.at[pl.ds(irow0 * _L, _C)], wsems[b]).wait()

    idx_start(0, 0)
    idx_wait(0)
    fire(0)
    idx_start(1, 1)
    drain(0)
    wb_start(0, 0)
    idx_wait(1)
    fire(1)
    idx_start(2, 0)

    @pl.loop(0, (G - 2) // 2)
    def _(i):
      for g_off, b in ((0, 1), (1, 0)):
        g = 1 + 2 * i + g_off
        ob = 1 - b
        drain(b)
        wb_start(g, b)
        idx_wait(ob)
        wb_wait(ob)
        fire(ob)
        idx_start(g + 2, b)

    drain(1)
    wb_start(G - 1, 1)
    idx_wait(0)
    wb_wait(0)
    wb_wait(1)

  return k


def kernel(x, table):
  B, H = x.shape
  V, D = table.shape
  N = B * H
  idx2d = x.reshape(N // _L, _L)
  out = _gather_fn(V, D, N)(table, idx2d)
  return out.reshape(B, H, D)
